# baseline (speedup 1.0000x reference)
import functools

import jax
import jax.numpy as jnp
from jax import lax
from jax.experimental import pallas as pl
from jax.experimental.pallas import tpu as pltpu
from jax.experimental.pallas import tpu_sc as plsc

NUM_EMB = 1_000_000
DIM = 64
NC = 2
NS = 16
NW = NC * NS
BATCH = 4096
SEQ = 200
BLK = 128
NBUF = 2
NGRP = SEQ // NBUF


def _emb_gather(pairs, idx_t):
    mesh = plsc.VectorSubcoreMesh(
        core_axis_name="c", subcore_axis_name="s", num_cores=NC, num_subcores=NS
    )

    @functools.partial(
        pl.kernel,
        out_type=jax.ShapeDtypeStruct((SEQ, DIM, BATCH), jnp.float32),
        mesh=mesh,
        compiler_params=pltpu.CompilerParams(needs_layout_passes=False),
        scratch_types=[
            pltpu.VMEM((SEQ, BLK), jnp.int32),
            pltpu.VMEM((NBUF, BLK), jnp.int32),
            pltpu.VMEM((NBUF, BLK), jnp.int32),
            [pltpu.VMEM((BLK, 128), jnp.float32) for _ in range(NBUF)],
            [pltpu.VMEM((DIM, BLK), jnp.float32) for _ in range(NBUF)],
            pltpu.SemaphoreType.DMA((NBUF,)),
            pltpu.SemaphoreType.DMA((NBUF,)),
        ],
    )
    def body(pairs_hbm, idx_hbm, out_hbm, idx_v, pidx, parv, pbufs, tbufs,
             gsem, osem):
        wid = lax.axis_index("s") * NC + lax.axis_index("c")
        lane0 = wid * BLK
        pltpu.sync_copy(idx_hbm.at[:, pl.ds(lane0, BLK)], idx_v)

        def fire(k, b):
            for j16 in range(BLK // 16):
                r = idx_v[k, pl.ds(j16 * 16, 16)]
                pidx[b, pl.ds(j16 * 16, 16)] = r >> 1
                parv[b, pl.ds(j16 * 16, 16)] = (r & 1) << 6
            pltpu.async_copy(pairs_hbm.at[pidx.at[b]], pbufs[b], gsem.at[b])

        def wait_gather(b):
            pltpu.make_async_copy(
                pairs_hbm.at[pidx.at[b]], pbufs[b], gsem.at[b]
            ).wait()

        def extract(b):
            iota = lax.iota(jnp.int32, 16)
            for j16 in range(BLK // 16):
                rowv = iota + (j16 * 16)
                pv = parv[b, pl.ds(j16 * 16, 16)]
                for c in range(DIM):
                    vals = plsc.load_gather(pbufs[b], [rowv, pv + c])
                    tbufs[b][c, pl.ds(j16 * 16, 16)] = vals

        for b in range(NBUF):
            fire(b, b)

        @pl.loop(0, NGRP - 1)
        def _(grp):
            for b in range(NBUF):
                k = grp * NBUF + b
                wait_gather(b)
                extract(b)
                pltpu.async_copy(
                    tbufs[b], out_hbm.at[k, :, pl.ds(lane0, BLK)], osem.at[b]
                )
            for b in range(NBUF):
                k = grp * NBUF + b
                pltpu.make_async_copy(
                    tbufs[b], out_hbm.at[k, :, pl.ds(lane0, BLK)], osem.at[b]
                ).wait()
                fire(k + NBUF, b)

        last = (NGRP - 1) * NBUF
        for b in range(NBUF):
            wait_gather(b)
            extract(b)
            pltpu.async_copy(
                tbufs[b], out_hbm.at[last + b, :, pl.ds(lane0, BLK)], osem.at[b]
            )
        for b in range(NBUF):
            pltpu.make_async_copy(
                tbufs[b], out_hbm.at[last + b, :, pl.ds(lane0, BLK)], osem.at[b]
            ).wait()

    return body(pairs, idx_t)


def kernel(embeddings, token_ids):
    pairs = embeddings.reshape(NUM_EMB // 2, 2 * DIM)
    idx_t = token_ids.astype(jnp.int32).T
    out = _emb_gather(pairs, idx_t)
    return out.transpose(2, 0, 1)


# speedup vs baseline: 2.6126x; 2.6126x over previous
import functools

import jax
import jax.numpy as jnp
from jax import lax
from jax.experimental import pallas as pl
from jax.experimental.pallas import tpu as pltpu
from jax.experimental.pallas import tpu_sc as plsc

NUM_EMB = 1_000_000
DIM = 64
NC = 2
NS = 16
NW = NC * NS
BATCH = 4096
SEQ = 200
BPW = BATCH // NW
NBUF = 2
NGRP = BPW // NBUF


def _emb_gather(table, idx):
    mesh = plsc.VectorSubcoreMesh(
        core_axis_name="c", subcore_axis_name="s", num_cores=NC, num_subcores=NS
    )

    @functools.partial(
        pl.kernel,
        out_type=jax.ShapeDtypeStruct((BATCH, SEQ, DIM), jnp.float32),
        mesh=mesh,
        scratch_types=[
            pltpu.VMEM((BPW, SEQ), jnp.int32),
            [pltpu.VMEM((SEQ, DIM), jnp.float32) for _ in range(NBUF)],
            pltpu.SemaphoreType.DMA((NBUF,)),
            pltpu.SemaphoreType.DMA((NBUF,)),
        ],
    )
    def body(table_hbm, idx_hbm, out_hbm, idx_v, bufs, gsem, osem):
        wid = lax.axis_index("s") * NC + lax.axis_index("c")
        base = wid * BPW
        pltpu.sync_copy(idx_hbm.at[pl.ds(base, BPW)], idx_v)

        def fire(local, b):
            for j16 in range(12):
                v = idx_v[local, pl.ds(j16 * 16, 16)]
                for j in range(16):
                    pltpu.async_copy(
                        table_hbm.at[v[j]], bufs[b].at[j16 * 16 + j], gsem.at[b]
                    )
            v = idx_v[local, pl.ds(SEQ - 16, 16)]
            for j in range(8, 16):
                pltpu.async_copy(
                    table_hbm.at[v[j]], bufs[b].at[SEQ - 16 + j], gsem.at[b]
                )

        def wait_gather(b):
            pltpu.make_async_copy(
                table_hbm.at[pl.ds(0, SEQ)], bufs[b], gsem.at[b]
            ).wait()

        for b in range(NBUF):
            fire(b, b)

        @pl.loop(0, NGRP - 1)
        def _(grp):
            for b in range(NBUF):
                local = grp * NBUF + b
                wait_gather(b)
                pltpu.async_copy(bufs[b], out_hbm.at[base + local], osem.at[b])
            for b in range(NBUF):
                local = grp * NBUF + b
                pltpu.make_async_copy(
                    bufs[b], out_hbm.at[base + local], osem.at[b]
                ).wait()
                fire(local + NBUF, b)

        last = (NGRP - 1) * NBUF
        for b in range(NBUF):
            wait_gather(b)
            pltpu.async_copy(bufs[b], out_hbm.at[base + last + b], osem.at[b])
        for b in range(NBUF):
            pltpu.make_async_copy(
                bufs[b], out_hbm.at[base + last + b], osem.at[b]
            ).wait()

    return body(table, idx)


def kernel(embeddings, token_ids):
    table_rm = jax.lax.optimization_barrier(embeddings.T).T
    return _emb_gather(table_rm, token_ids.astype(jnp.int32))
